# baseline (device time: 240166 ns/iter reference)
import jax
import jax.numpy as jnp
from jax import lax
from jax.experimental import pallas as pl
from jax.experimental.pallas import tpu as pltpu

N_CHUNKS = 32


def kernel(x):
    m, n = x.shape
    assert m % N_CHUNKS == 0
    rows = m // N_CHUNKS

    def body(x_ref, out_ref, rbuf, staging, local_sems,
             send_x, recv_x, send_y, recv_y):
        my_x = lax.axis_index("x")
        my_y = lax.axis_index("y")
        x_nbr = (1 - my_x, my_y)
        y_nbr = (my_x, 1 - my_y)
        col0 = my_y * n

        barrier_sem = pltpu.get_barrier_semaphore()
        for nbr in (x_nbr, y_nbr):
            pl.semaphore_signal(
                barrier_sem, inc=1,
                device_id=nbr, device_id_type=pl.DeviceIdType.MESH,
            )
        pl.semaphore_wait(barrier_sem, 2)

        def local_dma(c):
            return pltpu.make_async_copy(
                x_ref.at[pl.ds(c * rows, rows), :],
                staging.at[c % 2],
                local_sems.at[c % 2],
            )

        local_dma(0).start()
        local_dma(1).start()

        rdmas_x = []
        for c in range(N_CHUNKS):
            local_dma(c).wait()
            rsl = pl.ds(c * rows, rows)
            out_ref[rsl, pl.ds(col0, n)] = staging[c % 2].astype(jnp.bfloat16)
            if c + 2 < N_CHUNKS:
                local_dma(c + 2).start()
            r = pltpu.make_async_remote_copy(
                src_ref=out_ref.at[rsl, pl.ds(col0, n)],
                dst_ref=rbuf.at[rsl, :],
                send_sem=send_x.at[c],
                recv_sem=recv_x.at[c],
                device_id=x_nbr,
                device_id_type=pl.DeviceIdType.MESH,
            )
            r.start()
            rdmas_x.append(r)

        rdmas_y = []
        for c in range(N_CHUNKS):
            rdmas_x[c].wait_send()
            rdmas_x[c].wait_recv()
            rsl = pl.ds(c * rows, rows)
            out_ref[rsl, pl.ds(col0, n)] = (
                out_ref[rsl, pl.ds(col0, n)] + rbuf[rsl, :]
            )
            r = pltpu.make_async_remote_copy(
                src_ref=out_ref.at[rsl, pl.ds(col0, n)],
                dst_ref=out_ref.at[rsl, pl.ds(col0, n)],
                send_sem=send_y.at[c],
                recv_sem=recv_y.at[c],
                device_id=y_nbr,
                device_id_type=pl.DeviceIdType.MESH,
            )
            r.start()
            rdmas_y.append(r)

        for c in range(N_CHUNKS):
            rdmas_y[c].wait()

    return pl.pallas_call(
        body,
        out_shape=jax.ShapeDtypeStruct((m, 2 * n), jnp.bfloat16),
        in_specs=[pl.BlockSpec(memory_space=pl.ANY)],
        out_specs=pl.BlockSpec(memory_space=pltpu.VMEM),
        scratch_shapes=[
            pltpu.VMEM((m, n), jnp.bfloat16),
            pltpu.VMEM((2, m // N_CHUNKS, n), jnp.float32),
            pltpu.SemaphoreType.DMA((2,)),
            pltpu.SemaphoreType.DMA((N_CHUNKS,)),
            pltpu.SemaphoreType.DMA((N_CHUNKS,)),
            pltpu.SemaphoreType.DMA((N_CHUNKS,)),
            pltpu.SemaphoreType.DMA((N_CHUNKS,)),
        ],
        compiler_params=pltpu.CompilerParams(
            collective_id=0,
            vmem_limit_bytes=100 * 1024 * 1024,
        ),
    )(x)


# device time: 221748 ns/iter; 1.0831x vs baseline; 1.0831x over previous
import jax
import jax.numpy as jnp
from jax import lax
from jax.experimental import pallas as pl
from jax.experimental.pallas import tpu as pltpu

N_CHUNKS = 16


def kernel(x):
    m, n = x.shape
    assert m % N_CHUNKS == 0
    rows = m // N_CHUNKS

    def body(x_ref, out_ref, sbuf, rbuf, gbuf, staging, local_sems,
             send_x, recv_x, send_y, recv_y, out_sems_s, out_sems_g):
        my_x = lax.axis_index("x")
        my_y = lax.axis_index("y")
        x_nbr = (1 - my_x, my_y)
        y_nbr = (my_x, 1 - my_y)
        col_mine = my_y * n
        col_other = (1 - my_y) * n

        barrier_sem = pltpu.get_barrier_semaphore()
        for nbr in (x_nbr, y_nbr):
            pl.semaphore_signal(
                barrier_sem, inc=1,
                device_id=nbr, device_id_type=pl.DeviceIdType.MESH,
            )
        pl.semaphore_wait(barrier_sem, 2)

        def local_dma(c):
            return pltpu.make_async_copy(
                x_ref.at[pl.ds(c * rows, rows), :],
                staging.at[c % 2],
                local_sems.at[c % 2],
            )

        local_dma(0).start()
        local_dma(1).start()

        rdmas_x = []
        for c in range(N_CHUNKS):
            local_dma(c).wait()
            rsl = pl.ds(c * rows, rows)
            sbuf[rsl, :] = staging[c % 2].astype(jnp.bfloat16)
            if c + 2 < N_CHUNKS:
                local_dma(c + 2).start()
            r = pltpu.make_async_remote_copy(
                src_ref=sbuf.at[rsl, :],
                dst_ref=rbuf.at[rsl, :],
                send_sem=send_x.at[c],
                recv_sem=recv_x.at[c],
                device_id=x_nbr,
                device_id_type=pl.DeviceIdType.MESH,
            )
            r.start()
            rdmas_x.append(r)

        rdmas_y = []
        for c in range(N_CHUNKS):
            rdmas_x[c].wait_send()
            rdmas_x[c].wait_recv()
            rsl = pl.ds(c * rows, rows)
            sbuf[rsl, :] = sbuf[rsl, :] + rbuf[rsl, :]
            r = pltpu.make_async_remote_copy(
                src_ref=sbuf.at[rsl, :],
                dst_ref=gbuf.at[rsl, :],
                send_sem=send_y.at[c],
                recv_sem=recv_y.at[c],
                device_id=y_nbr,
                device_id_type=pl.DeviceIdType.MESH,
            )
            r.start()
            rdmas_y.append(r)
            pltpu.make_async_copy(
                sbuf.at[rsl, :],
                out_ref.at[rsl, pl.ds(col_mine, n)],
                out_sems_s.at[c],
            ).start()

        out_g = []
        for c in range(N_CHUNKS):
            rdmas_y[c].wait_send()
            rdmas_y[c].wait_recv()
            rsl = pl.ds(c * rows, rows)
            g = pltpu.make_async_copy(
                gbuf.at[rsl, :],
                out_ref.at[rsl, pl.ds(col_other, n)],
                out_sems_g.at[c],
            )
            g.start()
            out_g.append(g)

        for c in range(N_CHUNKS):
            rsl = pl.ds(c * rows, rows)
            pltpu.make_async_copy(
                sbuf.at[rsl, :],
                out_ref.at[rsl, pl.ds(col_mine, n)],
                out_sems_s.at[c],
            ).wait()
            out_g[c].wait()

    return pl.pallas_call(
        body,
        out_shape=jax.ShapeDtypeStruct((m, 2 * n), jnp.bfloat16),
        in_specs=[pl.BlockSpec(memory_space=pl.ANY)],
        out_specs=pl.BlockSpec(memory_space=pl.ANY),
        scratch_shapes=[
            pltpu.VMEM((m, n), jnp.bfloat16),
            pltpu.VMEM((m, n), jnp.bfloat16),
            pltpu.VMEM((m, n), jnp.bfloat16),
            pltpu.VMEM((2, m // N_CHUNKS, n), jnp.float32),
            pltpu.SemaphoreType.DMA((2,)),
            pltpu.SemaphoreType.DMA((N_CHUNKS,)),
            pltpu.SemaphoreType.DMA((N_CHUNKS,)),
            pltpu.SemaphoreType.DMA((N_CHUNKS,)),
            pltpu.SemaphoreType.DMA((N_CHUNKS,)),
            pltpu.SemaphoreType.DMA((N_CHUNKS,)),
            pltpu.SemaphoreType.DMA((N_CHUNKS,)),
        ],
        compiler_params=pltpu.CompilerParams(
            collective_id=0,
            vmem_limit_bytes=100 * 1024 * 1024,
        ),
    )(x)
